# baseline (speedup 1.0000x reference)
import functools

import jax
import jax.numpy as jnp
from jax import lax
from jax.experimental import pallas as pl
from jax.experimental.pallas import tpu as pltpu
from jax.experimental.pallas import tpu_sc as plsc

D_MODEL = 1024
N_TOKENS = 4 * 4096

_info = plsc.get_sparse_core_info()
NC, NS = _info.num_cores, _info.num_subcores
NW = NC * NS

CHUNK = 32
PER_W = N_TOKENS // NW
NCH = PER_W // CHUNK

_mesh = plsc.VectorSubcoreMesh(core_axis_name="c", subcore_axis_name="s")


@functools.partial(
    pl.kernel,
    out_type=jax.ShapeDtypeStruct((N_TOKENS, D_MODEL), jnp.float32),
    mesh=_mesh,
    scratch_types=[
        pltpu.VMEM((NCH, CHUNK), jnp.int32),
        pltpu.VMEM((CHUNK, D_MODEL), jnp.float32),
        pltpu.VMEM((CHUNK, D_MODEL), jnp.float32),
        pltpu.VMEM((CHUNK, D_MODEL), jnp.float32),
        pltpu.SemaphoreType.DMA,
        pltpu.SemaphoreType.DMA,
        pltpu.SemaphoreType.DMA,
        pltpu.SemaphoreType.DMA,
        pltpu.SemaphoreType.DMA,
        pltpu.SemaphoreType.DMA,
    ],
)
def _gather_kernel(idx_hbm, table_hbm, out_hbm, idx_v, rows0, rows1, rows2,
                   gs0, gs1, gs2, ws0, ws1, ws2):
    wid = lax.axis_index("s") * NC + lax.axis_index("c")
    pltpu.sync_copy(idx_hbm.at[wid], idx_v)
    base = wid * PER_W
    bufs, gsems, wsems = (rows0, rows1, rows2), (gs0, gs1, gs2), (ws0, ws1, ws2)

    def gather(i):
        b = i % 3
        return pltpu.make_async_copy(table_hbm.at[idx_v.at[i]], bufs[b],
                                     gsems[b])

    def write(i):
        b = i % 3
        return pltpu.make_async_copy(
            bufs[b], out_hbm.at[pl.ds(base + i * CHUNK, CHUNK)], wsems[b])

    gather(0).start()
    gather(1).start()
    for i in range(NCH):
        gather(i).wait()
        write(i).start()
        if i + 2 < NCH:
            if i >= 1:
                write(i - 1).wait()
            gather(i + 2).start()
    write(NCH - 2).wait()
    write(NCH - 1).wait()


def kernel(x, w_e):
    idx = x.reshape(NW, NCH, CHUNK).astype(jnp.int32)
    out = _gather_kernel(idx, w_e)
    return out.reshape(x.shape[0], x.shape[1], D_MODEL)


# speedup vs baseline: 1.0061x; 1.0061x over previous
import functools

import jax
import jax.numpy as jnp
from jax import lax
from jax.experimental import pallas as pl
from jax.experimental.pallas import tpu as pltpu
from jax.experimental.pallas import tpu_sc as plsc

D_MODEL = 1024
N_TOKENS = 4 * 4096

_info = plsc.get_sparse_core_info()
NC, NS = _info.num_cores, _info.num_subcores
NW = NC * NS

CHUNK = 32
PER_W = N_TOKENS // NW
NCH = PER_W // CHUNK

_mesh = plsc.VectorSubcoreMesh(core_axis_name="c", subcore_axis_name="s")


@functools.partial(
    pl.kernel,
    out_type=jax.ShapeDtypeStruct((N_TOKENS, D_MODEL), jnp.float32),
    mesh=_mesh,
    scratch_types=[
        pltpu.VMEM((PER_W,), jnp.int32),
        pltpu.VMEM((CHUNK, D_MODEL), jnp.float32),
        pltpu.VMEM((CHUNK, D_MODEL), jnp.float32),
        pltpu.SemaphoreType.DMA,
        pltpu.SemaphoreType.DMA,
        pltpu.SemaphoreType.DMA,
        pltpu.SemaphoreType.DMA,
    ],
)
def _gather_kernel(idx_hbm, table_hbm, out_hbm, idx_v, rows0, rows1,
                   gs0, gs1, ws0, ws1):
    wid = lax.axis_index("s") * NC + lax.axis_index("c")
    w_per_row = idx_hbm.shape[1] // PER_W
    pltpu.sync_copy(
        idx_hbm.at[wid // w_per_row, pl.ds((wid % w_per_row) * PER_W, PER_W)],
        idx_v)
    base = wid * PER_W
    bufs, gsems, wsems = (rows0, rows1), (gs0, gs1), (ws0, ws1)

    def gather(i):
        b = i % 2
        return pltpu.make_async_copy(
            table_hbm.at[idx_v.at[pl.ds(i * CHUNK, CHUNK)]], bufs[b],
            gsems[b])

    def write(i):
        b = i % 2
        return pltpu.make_async_copy(
            bufs[b], out_hbm.at[pl.ds(base + i * CHUNK, CHUNK)], wsems[b])

    gather(0).start()
    for i in range(NCH):
        if i + 1 < NCH:
            if i >= 1:
                write(i - 1).wait()
            gather(i + 1).start()
        gather(i).wait()
        write(i).start()
    write(NCH - 2).wait()
    write(NCH - 1).wait()


def kernel(x, w_e):
    out = _gather_kernel(x.astype(jnp.int32), w_e)
    return out.reshape(x.shape[0], x.shape[1], D_MODEL)
